# baseline (speedup 1.0000x reference)
import functools

import jax
import jax.numpy as jnp
from jax import lax
from jax.experimental import pallas as pl
from jax.experimental.pallas import tpu as pltpu
from jax.experimental.pallas import tpu_sc as plsc

VOCAB = 1000000
EMB = 64
SEQ = 200
BATCH = 16384

_NW = 32


_RCH = 512
_NFULL = VOCAB // _RCH
_TAIL = VOCAB - _NFULL * _RCH
_KMAX = -(-_NFULL // _NW)


def _sc_matvec_make():
    info = plsc.get_sparse_core_info()
    nc = info.num_cores
    mesh = plsc.VectorSubcoreMesh(core_axis_name="c", subcore_axis_name="s")

    @functools.partial(
        pl.kernel,
        mesh=mesh,
        out_type=jax.ShapeDtypeStruct((VOCAB,), jnp.float32),
        scratch_types=[
            pltpu.VMEM((_RCH, EMB), jnp.float32),
            pltpu.VMEM((_RCH,), jnp.float32),
            pltpu.VMEM((80,), jnp.float32),
        ],
    )
    def k(tbl_hbm, wb_hbm, p_hbm, chunk_v, pch_v, wb_v):
        wid = lax.axis_index("s") * nc + lax.axis_index("c")
        pltpu.sync_copy(wb_hbm, wb_v)
        w0 = wb_v[pl.ds(0, 16)]
        w1 = wb_v[pl.ds(16, 16)]
        w2 = wb_v[pl.ds(32, 16)]
        w3 = wb_v[pl.ds(48, 16)]
        bv = wb_v[pl.ds(64, 16)]

        lanes = lax.iota(jnp.int32, 16)
        onehot = [
            jnp.where(lanes == t, 1.0, 0.0).astype(jnp.float32) for t in range(16)
        ]
        _gdn = lax.GatherDimensionNumbers(
            offset_dims=(), collapsed_slice_dims=(0,), start_index_map=(0,)
        )
        perms = [(lanes ^ kk).reshape(16, 1) for kk in (1, 2, 4, 8)]

        def _hsum_all(v16):
            for pm in perms:
                v16 = v16 + lax.gather(
                    v16, pm, dimension_numbers=_gdn, slice_sizes=(1,),
                    mode=lax.GatherScatterMode.PROMISE_IN_BOUNDS,
                )
            return v16

        def process(row0, n):
            pltpu.sync_copy(tbl_hbm.at[pl.ds(row0, n), :], chunk_v.at[pl.ds(0, n), :])

            def grp(g, _):
                r0 = g * 16
                packed = jnp.zeros((16,), jnp.float32)
                for t in range(16):
                    r = r0 + t
                    v = (
                        chunk_v[r, pl.ds(0, 16)] * w0
                        + chunk_v[r, pl.ds(16, 16)] * w1
                        + chunk_v[r, pl.ds(32, 16)] * w2
                        + chunk_v[r, pl.ds(48, 16)] * w3
                        + bv
                    )
                    packed = packed + _hsum_all(v) * onehot[t]
                pch_v[pl.ds(r0, 16)] = packed
                return 0

            lax.fori_loop(0, n // 16, grp, 0)
            pltpu.sync_copy(pch_v.at[pl.ds(0, n)], p_hbm.at[pl.ds(row0, n)])

        def round_(kk, _):
            cid = wid + _NW * kk

            @pl.when(cid < _NFULL)
            def _():
                process(cid * _RCH, _RCH)

            return 0

        lax.fori_loop(0, _KMAX, round_, 0)

        @pl.when(wid == _NW - 1)
        def _():
            process(_NFULL * _RCH, _TAIL)

    return k


_sc_matvec = _sc_matvec_make()


_BPW = BATCH // _NW
_CS = 50
_CHUNK = _CS * _BPW


def _sc_gather_make():
    info = plsc.get_sparse_core_info()
    nc = info.num_cores
    mesh = plsc.VectorSubcoreMesh(core_axis_name="c", subcore_axis_name="s")

    @functools.partial(
        pl.kernel,
        mesh=mesh,
        out_type=jax.ShapeDtypeStruct((BATCH,), jnp.float32),
        scratch_types=[
            pltpu.VMEM((_CHUNK,), jnp.int32),
            pltpu.VMEM((_CHUNK,), jnp.float32),
            pltpu.VMEM((_BPW,), jnp.float32),
            pltpu.SemaphoreType.DMA,
            pltpu.SemaphoreType.DMA,
        ],
    )
    def k(p_hbm, xf_hbm, out_hbm, idx_v, vals_v, acc_v, lsem, gsem):
        wid = lax.axis_index("s") * nc + lax.axis_index("c")
        base = wid * _BPW
        for g in range(_BPW // 16):
            acc_v[pl.ds(g * 16, 16)] = jnp.zeros((16,), jnp.float32)
        for c in range(SEQ // _CS):
            def lrow(s, _):
                pltpu.async_copy(
                    xf_hbm.at[pl.ds((c * _CS + s) * BATCH + base, _BPW)],
                    idx_v.at[pl.ds(s * _BPW, _BPW)],
                    lsem,
                )
                return 0

            lax.fori_loop(0, _CS, lrow, 0)
            pltpu.make_async_copy(
                xf_hbm.at[pl.ds(0, _CHUNK)], idx_v, lsem
            ).wait()
            pltpu.async_copy(p_hbm.at[idx_v], vals_v, gsem).wait()

            def srow(s, _):
                for g in range(_BPW // 16):
                    acc_v[pl.ds(g * 16, 16)] += vals_v[
                        pl.ds(s * _BPW + g * 16, 16)
                    ]
                return 0

            lax.fori_loop(0, _CS, srow, 0)
        pltpu.sync_copy(acc_v, out_hbm.at[pl.ds(base, _BPW)])

    return k


_sc_gather_sum = _sc_gather_make()


def kernel(x, table, W, b):
    w = (W.astype(jnp.float32) / SEQ).reshape(EMB)
    bv = jnp.full((16,), b[0].astype(jnp.float32) / (SEQ * 16), jnp.float32)
    wb = jnp.concatenate([w, bv])
    p = _sc_matvec(table, wb)
    xf = x.reshape(SEQ * BATCH)
    return _sc_gather_sum(p, xf)
